# baseline (speedup 1.0000x reference)
import functools

import jax
import jax.numpy as jnp
from jax import lax
from jax.experimental import pallas as pl
from jax.experimental.pallas import tpu as pltpu
from jax.experimental.pallas import tpu_sc as plsc

EMBED_DIM = 64
BATCH = 16384
NUM_ENT = 1000000
_NB = (NUM_ENT + 127) // 128
_EDGE = _NB - 1
_EDGE_W = NUM_ENT - _EDGE * 128

_info = plsc.get_sparse_core_info()
_NC, _NS, _L = _info.num_cores, _info.num_subcores, _info.num_lanes
_NW = _NC * _NS
_BPW = BATCH // _NW
_BASE_BLKS = _NB // _NW
_EXTRA = _NB - _BASE_BLKS * _NW

_REC_CAP = 1536
_SEG = _REC_CAP // 4
_NCHUNK = _REC_CAP // 128
_NRING = 8
_STAGE_ROWS = 2 * BATCH + 128

_mesh = plsc.VectorSubcoreMesh(core_axis_name="c", subcore_axis_name="s")


@functools.partial(
    pl.kernel,
    mesh=_mesh,
    out_type=jax.ShapeDtypeStruct((_STAGE_ROWS, 128), jnp.float32),
    scratch_types=[
        pltpu.VMEM((8192,), jnp.int32),
        pltpu.VMEM((_REC_CAP + _L,), jnp.int32),
        pltpu.VMEM((_REC_CAP + _L,), jnp.int32),
        pltpu.VMEM((_REC_CAP + _L,), jnp.int32),
        pltpu.VMEM((_NCHUNK, 128), jnp.int32),
        pltpu.VMEM((4096,), jnp.int32),
        pltpu.VMEM((256 + _L,), jnp.int32),
        pltpu.VMEM((256 + _L,), jnp.int32),
        pltpu.VMEM((_NRING, EMBED_DIM, 128), jnp.float32),
        pltpu.VMEM((EMBED_DIM, _EDGE_W), jnp.float32),
        pltpu.VMEM((256, 128), jnp.float32),
        pltpu.SemaphoreType.DMA,
        pltpu.SemaphoreType.DMA,
    ],
    compiler_params=pltpu.CompilerParams(needs_layout_passes=False),
)
def _sweep(ent_t_hbm, edge_hbm, head_hbm, tail_hbm, stage_hbm,
           idbuf, fid, ftag, sid, stag2, hist, bstart, bcur,
           ring, eblk, stbuf, dsem, ssem):
    wid = lax.axis_index("s") * _NC + lax.axis_index("c")
    lo = wid * _BASE_BLKS + jnp.minimum(wid, _EXTRA)
    cnt = jnp.where(wid < _EXTRA, _BASE_BLKS + 1, _BASE_BLKS)
    lo_id = lo * 128
    hi_id = (lo + cnt) * 128
    lanes = lax.iota(jnp.int32, _L)
    ones = jnp.ones((_L,), jnp.int32)

    for k in range(_NRING):
        pltpu.make_async_copy(
            ent_t_hbm.at[:, pl.ds((lo + k) * 128, 128)], ring.at[k], dsem).start()

    @pl.when(lo + cnt - 1 == _EDGE)
    def _():
        pltpu.sync_copy(edge_hbm, eblk)

    chunks = ([(head_hbm, c * 2048, 0) for c in range(BATCH // 2048)]
              + [(tail_hbm, c * 2048, BATCH) for c in range(BATCH // 2048)])
    _scope_filter = jax.named_scope("p1_filter")
    _scope_filter.__enter__()

    for k in range(4):
        srck, basek, _ = chunks[k]
        pltpu.make_async_copy(
            srck.at[pl.ds(basek, 2048)], idbuf.at[pl.ds(k * 2048, 2048)], ssem).start()

    offs = tuple(q * _SEG for q in range(4))
    for ci, (src, hbase, tagoff) in enumerate(chunks):
        pltpu.make_async_copy(
            src.at[pl.ds(0, 2048)], idbuf.at[pl.ds(0, 2048)], ssem).wait()
        if ci + 4 < len(chunks):
            nsrc, nbase, _ = chunks[ci + 4]
            pltpu.make_async_copy(
                nsrc.at[pl.ds(nbase, 2048)],
                idbuf.at[pl.ds(((ci + 4) % 4) * 2048, 2048)], ssem).start()
        pbase = (ci % 4) * 2048
        tagbase = tagoff + hbase

        def g_body(g, offs, pbase=pbase, tagbase=tagbase):
            new = []
            for q in range(4):
                ids = idbuf[pl.ds(pbase + q * 512 + g * _L, _L)]
                m = (ids >= lo_id) & (ids < hi_id)
                mi = m.astype(jnp.int32)
                s = plsc.cumsum(mi)
                pos = jnp.clip(offs[q] + s - 1, q * _SEG, (q + 1) * _SEG - _L)
                plsc.store_scatter(fid, [pos], ids, mask=m)
                tags = tagbase + q * 512 + g * _L + lanes
                plsc.store_scatter(ftag, [pos], tags, mask=m)
                new.append(jnp.minimum(offs[q] + s[_L - 1],
                                       (q + 1) * _SEG - _L))
            return tuple(new)

        offs = lax.fori_loop(0, 512 // _L, g_body, offs, unroll=2)
    nrec = (offs[0] + offs[1] + offs[2] + offs[3]
            - (_SEG + 2 * _SEG + 3 * _SEG))
    _scope_filter.__exit__(None, None, None)
    _scope_sort = jax.named_scope("p2_sort")
    _scope_sort.__enter__()

    for q in range(4):
        fid[pl.ds(offs[q], _L)] = jnp.zeros((_L,), jnp.int32) + (lo + 255) * 128

    def z_body(i, c):
        hist[pl.ds(i * _L, _L)] = jnp.zeros((_L,), jnp.int32)
        return c

    lax.fori_loop(0, 4096 // _L, z_body, 0)

    for q in range(4):
        def h_body(g, c, q=q):
            ids = fid[pl.ds(q * _SEG + g * _L, _L)]
            jb = (ids >> 7) - lo
            plsc.addupdate_scatter(hist, [jb * _L + lanes], ones)
            return c

        lax.fori_loop(0, (offs[q] - q * _SEG + _L - 1) // _L, h_body, 0)

    def p_body(k, run):
        bins = k * _L + lanes
        tot = jnp.zeros((_L,), jnp.int32)
        for l in range(_L):
            tot = tot + plsc.load_gather(hist, [bins * _L + l])
        s = plsc.cumsum(tot)
        excl = s - tot + run
        bstart[pl.ds(k * _L, _L)] = excl
        bcur[pl.ds(k * _L, _L)] = excl
        return run + s[_L - 1]

    lax.fori_loop(0, 256 // _L, p_body, 0)

    def f_body(i, c):
        plsc.store_scatter(stag2, [jnp.zeros((_L,), jnp.int32) + (i >> 3),
                                   (i & 7) * _L + lanes],
                           jnp.zeros((_L,), jnp.int32) + 2 * BATCH)
        return c

    lax.fori_loop(0, _NCHUNK * 8, f_body, 0)

    lane0 = lanes == 0

    def s_body(r, c):
        idv = fid[pl.ds(r, _L)][0]
        tgv = ftag[pl.ds(r, _L)][0]
        jb = (idv >> 7) - lo
        dst = bcur[pl.ds(jb, _L)][0]
        dstv = jnp.zeros((_L,), jnp.int32) + dst
        plsc.store_scatter(sid, [dstv], jnp.zeros((_L,), jnp.int32) + idv, mask=lane0)
        plsc.store_scatter(stag2, [jnp.zeros((_L,), jnp.int32) + (dst >> 7),
                                   jnp.zeros((_L,), jnp.int32) + (dst & 127)],
                           jnp.zeros((_L,), jnp.int32) + tgv, mask=lane0)
        plsc.store_scatter(bcur, [jnp.zeros((_L,), jnp.int32) + jb],
                           dstv + 1, mask=lane0)
        return c

    for q in range(4):
        lax.fori_loop(q * _SEG, offs[q], s_body, 0)
    _scope_sort.__exit__(None, None, None)
    _scope_sweep = jax.named_scope("p3_sweep")
    _scope_sweep.__enter__()

    def make_rec_body(gather_cols):
        def rec_body(r, c):
            @pl.when(((r & 127) == 0) & ((r >> 7) >= 2))
            def _():
                pltpu.make_async_copy(
                    stbuf.at[pl.ds(0, 128)], stage_hbm.at[stag2.at[0]], ssem).wait()

            idv = sid[pl.ds(r, _L)][0]
            rrv = jnp.zeros((_L,), jnp.int32) + (idv & 127)
            sbv = jnp.zeros((_L,), jnp.int32) + (r & 255)
            for k in range(EMBED_DIM // _L):
                v = gather_cols(k * _L + lanes, rrv)
                plsc.store_scatter(stbuf, [sbv, k * _L + lanes], v)

            @pl.when((r & 127) == 127)
            def _():
                ch = r >> 7
                pltpu.make_async_copy(
                    stbuf.at[pl.ds((ch & 1) * 128, 128)],
                    stage_hbm.at[stag2.at[ch]], ssem).start()

            return c

        return rec_body

    def blk_body(j, c):
        gb = lo + j
        is_edge = gb == _EDGE
        slot = j & (_NRING - 1)
        bsv = bstart[pl.ds(j, _L)]

        @pl.when(is_edge)
        def _():
            lax.fori_loop(bsv[0], bsv[1], make_rec_body(
                lambda dv, rrv: plsc.load_gather(eblk, [dv, rrv])), 0)

        @pl.when(jnp.logical_not(is_edge))
        def _():
            pltpu.make_async_copy(
                ent_t_hbm.at[:, pl.ds(0, 128)], ring.at[0], dsem).wait()
            slotv = jnp.zeros((_L,), jnp.int32) + slot
            lax.fori_loop(bsv[0], bsv[1], make_rec_body(
                lambda dv, rrv: plsc.load_gather(ring, [slotv, dv, rrv])), 0)

        gb2 = lo + j + _NRING

        @pl.when((j + _NRING < cnt) & (gb2 != _EDGE))
        def _():
            pltpu.make_async_copy(
                ent_t_hbm.at[:, pl.ds(gb2 * 128, 128)],
                ring.at[slot], dsem).start()

        return c

    lax.fori_loop(0, cnt, blk_body, 0)

    @pl.when((nrec & 127) != 0)
    def _():
        ch = nrec >> 7
        pltpu.make_async_copy(
            stbuf.at[pl.ds((ch & 1) * 128, 128)],
            stage_hbm.at[stag2.at[ch]], ssem).start()

    total_fired = (nrec + 127) >> 7
    drained = jnp.maximum(((nrec - 1) >> 7) - 1, 0)

    def d_body(i, c):
        pltpu.make_async_copy(
            stbuf.at[pl.ds(0, 128)], stage_hbm.at[stag2.at[0]], ssem).wait()
        return c

    lax.fori_loop(0, total_fired - drained, d_body, 0)
    _scope_sweep.__exit__(None, None, None)


@functools.partial(
    pl.kernel,
    mesh=_mesh,
    out_type=jax.ShapeDtypeStruct((BATCH,), jnp.float32),
    scratch_types=[
        pltpu.VMEM((8, EMBED_DIM, 128), jnp.float32),
        pltpu.VMEM((512,), jnp.int32),
        pltpu.VMEM((128, 128), jnp.float32),
        pltpu.VMEM((128, 128), jnp.float32),
        pltpu.VMEM((_BPW,), jnp.float32),
        pltpu.SemaphoreType.DMA,
    ],
    compiler_params=pltpu.CompilerParams(needs_layout_passes=False),
)
def _score(stage_hbm, rel_t_hbm, relidx_hbm, out_hbm,
           relbuf, ridx, hbuf, tbuf, outv, sem):
    wid = lax.axis_index("s") * _NC + lax.axis_index("c")
    base = wid * _BPW
    lanes = lax.iota(jnp.int32, _L)

    copies = [pltpu.async_copy(rel_t_hbm.at[:, pl.ds(k * 128, 128)],
                               relbuf.at[k], sem) for k in range(8)]
    pltpu.sync_copy(relidx_hbm.at[pl.ds(base, _BPW)], ridx)

    npass = _BPW // 128
    for c in copies:
        c.wait()

    for p in range(npass):
        ch = pltpu.async_copy(stage_hbm.at[pl.ds(base + p * 128, 128)], hbuf, sem)
        ct = pltpu.async_copy(
            stage_hbm.at[pl.ds(BATCH + base + p * 128, 128)], tbuf, sem)
        ch.wait()
        ct.wait()

        def g_body(g, c, p=p):
            relv = ridx[pl.ds(p * 128 + g * _L, _L)]
            jv = relv >> 7
            rv = relv & 127
            row = g * _L + lanes

            def d_body(d, acc):
                dd = jnp.zeros((_L,), jnp.int32) + d
                h = plsc.load_gather(hbuf, [row, dd])
                t = plsc.load_gather(tbuf, [row, dd])
                r = plsc.load_gather(relbuf, [jv, dd, rv])
                return acc + jnp.abs(h + r - t)

            acc = lax.fori_loop(0, EMBED_DIM, d_body,
                                jnp.zeros((_L,), jnp.float32), unroll=8)
            outv[pl.ds(p * 128 + g * _L, _L)] = acc
            return c

        lax.fori_loop(0, 128 // _L, g_body, 0)

    pltpu.sync_copy(outv, out_hbm.at[pl.ds(base, _BPW)])


def kernel(entity_emb, relation_emb, head, relation, tail):
    ent_t = entity_emb.T
    ent_edge = entity_emb[_EDGE * 128:].T
    rel_t = jnp.pad(relation_emb, ((0, 24), (0, 0))).T
    staging = _sweep(ent_t, ent_edge,
                     head.astype(jnp.int32), tail.astype(jnp.int32))
    return _score(staging, rel_t, relation.astype(jnp.int32))


# speedup vs baseline: 1.0476x; 1.0476x over previous
import functools

import jax
import jax.numpy as jnp
from jax import lax
from jax.experimental import pallas as pl
from jax.experimental.pallas import tpu as pltpu
from jax.experimental.pallas import tpu_sc as plsc

EMBED_DIM = 64
BATCH = 16384
NUM_ENT = 1000000
_NB = (NUM_ENT + 127) // 128
_EDGE = _NB - 1
_EDGE_W = NUM_ENT - _EDGE * 128

_info = plsc.get_sparse_core_info()
_NC, _NS, _L = _info.num_cores, _info.num_subcores, _info.num_lanes
_NW = _NC * _NS
_BPW = BATCH // _NW
_BASE_BLKS = _NB // _NW
_EXTRA = _NB - _BASE_BLKS * _NW

_REC_CAP = 1536
_SEG = _REC_CAP // 4
_NCHUNK = _REC_CAP // 128
_NRING = 8
_STAGE_ROWS = 2 * BATCH + 128

_mesh = plsc.VectorSubcoreMesh(core_axis_name="c", subcore_axis_name="s")


@functools.partial(
    pl.kernel,
    mesh=_mesh,
    out_type=jax.ShapeDtypeStruct((_STAGE_ROWS, 128), jnp.float32),
    scratch_types=[
        pltpu.VMEM((8192,), jnp.int32),
        pltpu.VMEM((_REC_CAP + _L,), jnp.int32),
        pltpu.VMEM((_REC_CAP + _L,), jnp.int32),
        pltpu.VMEM((_REC_CAP + _L,), jnp.int32),
        pltpu.VMEM((_NCHUNK, 128), jnp.int32),
        pltpu.VMEM((4096,), jnp.int32),
        pltpu.VMEM((256 + _L,), jnp.int32),
        pltpu.VMEM((256 + _L,), jnp.int32),
        pltpu.VMEM((_NRING, EMBED_DIM, 128), jnp.float32),
        pltpu.VMEM((EMBED_DIM, _EDGE_W), jnp.float32),
        pltpu.VMEM((256, 128), jnp.float32),
        pltpu.SemaphoreType.DMA,
        pltpu.SemaphoreType.DMA,
    ],
    compiler_params=pltpu.CompilerParams(needs_layout_passes=False),
)
def _sweep(ent_t_hbm, edge_hbm, head_hbm, tail_hbm, stage_hbm,
           idbuf, fid, ftag, sid, stag2, hist, bstart, bcur,
           ring, eblk, stbuf, dsem, ssem):
    wid = lax.axis_index("s") * _NC + lax.axis_index("c")
    lo = wid * _BASE_BLKS + jnp.minimum(wid, _EXTRA)
    cnt = jnp.where(wid < _EXTRA, _BASE_BLKS + 1, _BASE_BLKS)
    lo_id = lo * 128
    hi_id = (lo + cnt) * 128
    lanes = lax.iota(jnp.int32, _L)
    ones = jnp.ones((_L,), jnp.int32)

    for k in range(_NRING):
        pltpu.make_async_copy(
            ent_t_hbm.at[:, pl.ds((lo + k) * 128, 128)], ring.at[k], dsem).start()

    @pl.when(lo + cnt - 1 == _EDGE)
    def _():
        pltpu.sync_copy(edge_hbm, eblk)

    chunks = ([(head_hbm, c * 2048, 0) for c in range(BATCH // 2048)]
              + [(tail_hbm, c * 2048, BATCH) for c in range(BATCH // 2048)])
    _scope_filter = jax.named_scope("p1_filter")
    _scope_filter.__enter__()

    for k in range(4):
        srck, basek, _ = chunks[k]
        pltpu.make_async_copy(
            srck.at[pl.ds(basek, 2048)], idbuf.at[pl.ds(k * 2048, 2048)], ssem).start()

    offs = tuple(q * _SEG for q in range(4))
    for ci, (src, hbase, tagoff) in enumerate(chunks):
        pltpu.make_async_copy(
            src.at[pl.ds(0, 2048)], idbuf.at[pl.ds(0, 2048)], ssem).wait()
        pbase = (ci % 4) * 2048
        tagbase = tagoff + hbase

        def g_body(g, offs, pbase=pbase, tagbase=tagbase):
            new = []
            for q in range(4):
                ids = idbuf[pl.ds(pbase + q * 512 + g * _L, _L)]
                m = (ids >= lo_id) & (ids < hi_id)
                mi = m.astype(jnp.int32)
                s = plsc.cumsum(mi)
                pos = jnp.clip(offs[q] + s - 1, q * _SEG, (q + 1) * _SEG - _L)
                plsc.store_scatter(fid, [pos], ids, mask=m)
                tags = tagbase + q * 512 + g * _L + lanes
                plsc.store_scatter(ftag, [pos], tags, mask=m)
                new.append(jnp.minimum(offs[q] + s[_L - 1],
                                       (q + 1) * _SEG - _L))
            return tuple(new)

        offs = lax.fori_loop(0, 512 // _L, g_body, offs, unroll=2)
        if ci + 4 < len(chunks):
            nsrc, nbase, _ = chunks[ci + 4]
            pltpu.make_async_copy(
                nsrc.at[pl.ds(nbase, 2048)],
                idbuf.at[pl.ds(((ci + 4) % 4) * 2048, 2048)], ssem).start()
    nrec = (offs[0] + offs[1] + offs[2] + offs[3]
            - (_SEG + 2 * _SEG + 3 * _SEG))
    _scope_filter.__exit__(None, None, None)
    _scope_sort = jax.named_scope("p2_sort")
    _scope_sort.__enter__()

    for q in range(4):
        fid[pl.ds(offs[q], _L)] = jnp.zeros((_L,), jnp.int32) + (lo + 255) * 128

    def z_body(i, c):
        hist[pl.ds(i * _L, _L)] = jnp.zeros((_L,), jnp.int32)
        return c

    lax.fori_loop(0, 4096 // _L, z_body, 0)

    for q in range(4):
        def h_body(g, c, q=q):
            ids = fid[pl.ds(q * _SEG + g * _L, _L)]
            jb = (ids >> 7) - lo
            plsc.addupdate_scatter(hist, [jb * _L + lanes], ones)
            return c

        lax.fori_loop(0, (offs[q] - q * _SEG + _L - 1) // _L, h_body, 0)

    def p_body(k, run):
        bins = k * _L + lanes
        tot = jnp.zeros((_L,), jnp.int32)
        for l in range(_L):
            tot = tot + plsc.load_gather(hist, [bins * _L + l])
        s = plsc.cumsum(tot)
        excl = s - tot + run
        bstart[pl.ds(k * _L, _L)] = excl
        bcur[pl.ds(k * _L, _L)] = excl
        return run + s[_L - 1]

    lax.fori_loop(0, 256 // _L, p_body, 0)

    def f_body(i, c):
        plsc.store_scatter(stag2, [jnp.zeros((_L,), jnp.int32) + (i >> 3),
                                   (i & 7) * _L + lanes],
                           jnp.zeros((_L,), jnp.int32) + 2 * BATCH)
        return c

    lax.fori_loop(0, _NCHUNK * 8, f_body, 0)

    lane0 = lanes == 0

    def s_body(r, c):
        idv = fid[pl.ds(r, _L)][0]
        tgv = ftag[pl.ds(r, _L)][0]
        jb = (idv >> 7) - lo
        dst = bcur[pl.ds(jb, _L)][0]
        dstv = jnp.zeros((_L,), jnp.int32) + dst
        plsc.store_scatter(sid, [dstv], jnp.zeros((_L,), jnp.int32) + idv, mask=lane0)
        plsc.store_scatter(stag2, [jnp.zeros((_L,), jnp.int32) + (dst >> 7),
                                   jnp.zeros((_L,), jnp.int32) + (dst & 127)],
                           jnp.zeros((_L,), jnp.int32) + tgv, mask=lane0)
        plsc.store_scatter(bcur, [jnp.zeros((_L,), jnp.int32) + jb],
                           dstv + 1, mask=lane0)
        return c

    for q in range(4):
        lax.fori_loop(q * _SEG, offs[q], s_body, 0)
    _scope_sort.__exit__(None, None, None)
    _scope_sweep = jax.named_scope("p3_sweep")
    _scope_sweep.__enter__()

    def make_rec_body(gather_cols):
        def rec_body(r, c):
            @pl.when(((r & 127) == 0) & ((r >> 7) >= 2))
            def _():
                pltpu.make_async_copy(
                    stbuf.at[pl.ds(0, 128)], stage_hbm.at[stag2.at[0]], ssem).wait()

            idv = sid[pl.ds(r, _L)][0]
            rrv = jnp.zeros((_L,), jnp.int32) + (idv & 127)
            sbv = jnp.zeros((_L,), jnp.int32) + (r & 255)
            for k in range(EMBED_DIM // _L):
                v = gather_cols(k * _L + lanes, rrv)
                plsc.store_scatter(stbuf, [sbv, k * _L + lanes], v)

            @pl.when((r & 127) == 127)
            def _():
                ch = r >> 7
                pltpu.make_async_copy(
                    stbuf.at[pl.ds((ch & 1) * 128, 128)],
                    stage_hbm.at[stag2.at[ch]], ssem).start()

            return c

        return rec_body

    def blk_body(j, c):
        gb = lo + j
        is_edge = gb == _EDGE
        slot = j & (_NRING - 1)
        bsv = bstart[pl.ds(j, _L)]

        @pl.when(is_edge)
        def _():
            lax.fori_loop(bsv[0], bsv[1], make_rec_body(
                lambda dv, rrv: plsc.load_gather(eblk, [dv, rrv])), 0)

        @pl.when(jnp.logical_not(is_edge))
        def _():
            pltpu.make_async_copy(
                ent_t_hbm.at[:, pl.ds(0, 128)], ring.at[0], dsem).wait()
            slotv = jnp.zeros((_L,), jnp.int32) + slot
            lax.fori_loop(bsv[0], bsv[1], make_rec_body(
                lambda dv, rrv: plsc.load_gather(ring, [slotv, dv, rrv])), 0)

        gb2 = lo + j + _NRING

        @pl.when((j + _NRING < cnt) & (gb2 != _EDGE))
        def _():
            pltpu.make_async_copy(
                ent_t_hbm.at[:, pl.ds(gb2 * 128, 128)],
                ring.at[slot], dsem).start()

        return c

    lax.fori_loop(0, cnt, blk_body, 0)

    @pl.when((nrec & 127) != 0)
    def _():
        ch = nrec >> 7
        pltpu.make_async_copy(
            stbuf.at[pl.ds((ch & 1) * 128, 128)],
            stage_hbm.at[stag2.at[ch]], ssem).start()

    total_fired = (nrec + 127) >> 7
    drained = jnp.maximum(((nrec - 1) >> 7) - 1, 0)

    def d_body(i, c):
        pltpu.make_async_copy(
            stbuf.at[pl.ds(0, 128)], stage_hbm.at[stag2.at[0]], ssem).wait()
        return c

    lax.fori_loop(0, total_fired - drained, d_body, 0)
    _scope_sweep.__exit__(None, None, None)


@functools.partial(
    pl.kernel,
    mesh=_mesh,
    out_type=jax.ShapeDtypeStruct((BATCH,), jnp.float32),
    scratch_types=[
        pltpu.VMEM((8, EMBED_DIM, 128), jnp.float32),
        pltpu.VMEM((512,), jnp.int32),
        pltpu.VMEM((128, 128), jnp.float32),
        pltpu.VMEM((128, 128), jnp.float32),
        pltpu.VMEM((_BPW,), jnp.float32),
        pltpu.SemaphoreType.DMA,
    ],
    compiler_params=pltpu.CompilerParams(needs_layout_passes=False),
)
def _score(stage_hbm, rel_t_hbm, relidx_hbm, out_hbm,
           relbuf, ridx, hbuf, tbuf, outv, sem):
    wid = lax.axis_index("s") * _NC + lax.axis_index("c")
    base = wid * _BPW
    lanes = lax.iota(jnp.int32, _L)

    copies = [pltpu.async_copy(rel_t_hbm.at[:, pl.ds(k * 128, 128)],
                               relbuf.at[k], sem) for k in range(8)]
    pltpu.sync_copy(relidx_hbm.at[pl.ds(base, _BPW)], ridx)

    npass = _BPW // 128
    for c in copies:
        c.wait()

    for p in range(npass):
        ch = pltpu.async_copy(stage_hbm.at[pl.ds(base + p * 128, 128)], hbuf, sem)
        ct = pltpu.async_copy(
            stage_hbm.at[pl.ds(BATCH + base + p * 128, 128)], tbuf, sem)
        ch.wait()
        ct.wait()

        def g_body(g, c, p=p):
            relv = ridx[pl.ds(p * 128 + g * _L, _L)]
            jv = relv >> 7
            rv = relv & 127
            row = g * _L + lanes

            def d_body(d, acc):
                dd = jnp.zeros((_L,), jnp.int32) + d
                h = plsc.load_gather(hbuf, [row, dd])
                t = plsc.load_gather(tbuf, [row, dd])
                r = plsc.load_gather(relbuf, [jv, dd, rv])
                return acc + jnp.abs(h + r - t)

            acc = lax.fori_loop(0, EMBED_DIM, d_body,
                                jnp.zeros((_L,), jnp.float32), unroll=8)
            outv[pl.ds(p * 128 + g * _L, _L)] = acc
            return c

        lax.fori_loop(0, 128 // _L, g_body, 0)

    pltpu.sync_copy(outv, out_hbm.at[pl.ds(base, _BPW)])


def kernel(entity_emb, relation_emb, head, relation, tail):
    ent_t = entity_emb.T
    ent_edge = entity_emb[_EDGE * 128:].T
    rel_t = jnp.pad(relation_emb, ((0, 24), (0, 0))).T
    staging = _sweep(ent_t, ent_edge,
                     head.astype(jnp.int32), tail.astype(jnp.int32))
    return _score(staging, rel_t, relation.astype(jnp.int32))


# speedup vs baseline: 1.0677x; 1.0192x over previous
import functools

import jax
import jax.numpy as jnp
from jax import lax
from jax.experimental import pallas as pl
from jax.experimental.pallas import tpu as pltpu
from jax.experimental.pallas import tpu_sc as plsc

EMBED_DIM = 64
BATCH = 16384
NUM_ENT = 1000000
_NB = (NUM_ENT + 127) // 128
_EDGE = _NB - 1
_EDGE_W = NUM_ENT - _EDGE * 128

_info = plsc.get_sparse_core_info()
_NC, _NS, _L = _info.num_cores, _info.num_subcores, _info.num_lanes
_NW = _NC * _NS
_BPW = BATCH // _NW
_BASE_BLKS = _NB // _NW
_EXTRA = _NB - _BASE_BLKS * _NW

_REC_CAP = 1536
_SEG = _REC_CAP // 4
_NCHUNK = _REC_CAP // 128
_NRING = 8
_STAGE_ROWS = 2 * BATCH + 128

_mesh = plsc.VectorSubcoreMesh(core_axis_name="c", subcore_axis_name="s")


@functools.partial(
    pl.kernel,
    mesh=_mesh,
    out_type=jax.ShapeDtypeStruct((_STAGE_ROWS, 128), jnp.float32),
    scratch_types=[
        pltpu.VMEM((8192,), jnp.int32),
        pltpu.VMEM((_REC_CAP + _L,), jnp.int32),
        pltpu.VMEM((_REC_CAP + _L,), jnp.int32),
        pltpu.VMEM((_REC_CAP + _L,), jnp.int32),
        pltpu.VMEM((_NCHUNK, 128), jnp.int32),
        pltpu.VMEM((4096,), jnp.int32),
        pltpu.VMEM((256 + _L,), jnp.int32),
        pltpu.VMEM((256 + _L,), jnp.int32),
        pltpu.VMEM((_NRING, EMBED_DIM, 128), jnp.float32),
        pltpu.VMEM((EMBED_DIM, _EDGE_W), jnp.float32),
        pltpu.VMEM((256, 128), jnp.float32),
        pltpu.SemaphoreType.DMA,
        pltpu.SemaphoreType.DMA,
    ],
    compiler_params=pltpu.CompilerParams(needs_layout_passes=False),
)
def _sweep(ent_t_hbm, edge_hbm, head_hbm, tail_hbm, stage_hbm,
           idbuf, fid, ftag, sid, stag2, hist, bstart, bcur,
           ring, eblk, stbuf, dsem, ssem):
    wid = lax.axis_index("s") * _NC + lax.axis_index("c")
    lo = wid * _BASE_BLKS + jnp.minimum(wid, _EXTRA)
    cnt = jnp.where(wid < _EXTRA, _BASE_BLKS + 1, _BASE_BLKS)
    lo_id = lo * 128
    hi_id = (lo + cnt) * 128
    lanes = lax.iota(jnp.int32, _L)
    ones = jnp.ones((_L,), jnp.int32)

    for k in range(_NRING):
        pltpu.make_async_copy(
            ent_t_hbm.at[:, pl.ds((lo + k) * 128, 128)], ring.at[k], dsem).start()

    @pl.when(lo + cnt - 1 == _EDGE)
    def _():
        pltpu.sync_copy(edge_hbm, eblk)

    chunks = ([(head_hbm, c * 2048, 0) for c in range(BATCH // 2048)]
              + [(tail_hbm, c * 2048, BATCH) for c in range(BATCH // 2048)])
    _scope_filter = jax.named_scope("p1_filter")
    _scope_filter.__enter__()

    for k in range(4):
        srck, basek, _ = chunks[k]
        pltpu.make_async_copy(
            srck.at[pl.ds(basek, 2048)], idbuf.at[pl.ds(k * 2048, 2048)], ssem).start()

    offs = tuple(q * _SEG for q in range(4))
    for ci, (src, hbase, tagoff) in enumerate(chunks):
        pltpu.make_async_copy(
            src.at[pl.ds(0, 2048)], idbuf.at[pl.ds(0, 2048)], ssem).wait()
        pbase = (ci % 4) * 2048
        tagbase = tagoff + hbase

        def g_body(g, offs, pbase=pbase, tagbase=tagbase):
            new = []
            for q in range(4):
                ids = idbuf[pl.ds(pbase + q * 512 + g * _L, _L)]
                m = (ids >= lo_id) & (ids < hi_id)
                mi = m.astype(jnp.int32)
                s = plsc.cumsum(mi)
                pos = jnp.clip(offs[q] + s - 1, q * _SEG, (q + 1) * _SEG - _L)
                plsc.store_scatter(fid, [pos], ids, mask=m)
                tags = tagbase + q * 512 + g * _L + lanes
                plsc.store_scatter(ftag, [pos], tags, mask=m)
                new.append(jnp.minimum(offs[q] + s[_L - 1],
                                       (q + 1) * _SEG - _L))
            return tuple(new)

        offs = lax.fori_loop(0, 512 // _L, g_body, offs, unroll=2)
        if ci + 4 < len(chunks):
            nsrc, nbase, _ = chunks[ci + 4]
            pltpu.make_async_copy(
                nsrc.at[pl.ds(nbase, 2048)],
                idbuf.at[pl.ds(((ci + 4) % 4) * 2048, 2048)], ssem).start()
    nrec = (offs[0] + offs[1] + offs[2] + offs[3]
            - (_SEG + 2 * _SEG + 3 * _SEG))
    _scope_filter.__exit__(None, None, None)
    _scope_sort = jax.named_scope("p2_sort")
    _scope_sort.__enter__()

    for q in range(4):
        fid[pl.ds(offs[q], _L)] = jnp.zeros((_L,), jnp.int32) + (lo + 255) * 128

    def z_body(i, c):
        hist[pl.ds(i * _L, _L)] = jnp.zeros((_L,), jnp.int32)
        return c

    lax.fori_loop(0, 4096 // _L, z_body, 0)

    for q in range(4):
        def h_body(g, c, q=q):
            ids = fid[pl.ds(q * _SEG + g * _L, _L)]
            jb = (ids >> 7) - lo
            plsc.addupdate_scatter(hist, [jb * _L + lanes], ones)
            return c

        lax.fori_loop(0, (offs[q] - q * _SEG + _L - 1) // _L, h_body, 0)

    def p_body(k, run):
        bins = k * _L + lanes
        tot = jnp.zeros((_L,), jnp.int32)
        for l in range(_L):
            tot = tot + plsc.load_gather(hist, [bins * _L + l])
        s = plsc.cumsum(tot)
        excl = s - tot + run
        bstart[pl.ds(k * _L, _L)] = excl
        bcur[pl.ds(k * _L, _L)] = excl
        return run + s[_L - 1]

    lax.fori_loop(0, 256 // _L, p_body, 0)

    def f_body(i, c):
        plsc.store_scatter(stag2, [jnp.zeros((_L,), jnp.int32) + (i >> 3),
                                   (i & 7) * _L + lanes],
                           jnp.zeros((_L,), jnp.int32) + 2 * BATCH)
        return c

    lax.fori_loop(0, _NCHUNK * 8, f_body, 0)

    lane0 = lanes == 0

    def s_body(r, c):
        idv = fid[pl.ds(r, _L)][0]
        tgv = ftag[pl.ds(r, _L)][0]
        jb = (idv >> 7) - lo
        dst = bcur[pl.ds(jb, _L)][0]
        dstv = jnp.zeros((_L,), jnp.int32) + dst
        plsc.store_scatter(sid, [dstv], jnp.zeros((_L,), jnp.int32) + idv, mask=lane0)
        plsc.store_scatter(stag2, [jnp.zeros((_L,), jnp.int32) + (dst >> 7),
                                   jnp.zeros((_L,), jnp.int32) + (dst & 127)],
                           jnp.zeros((_L,), jnp.int32) + tgv, mask=lane0)
        plsc.store_scatter(bcur, [jnp.zeros((_L,), jnp.int32) + jb],
                           dstv + 1, mask=lane0)
        return c

    for q in range(4):
        lax.fori_loop(q * _SEG, offs[q], s_body, 0)
    _scope_sort.__exit__(None, None, None)
    _scope_sweep = jax.named_scope("p3_sweep")
    _scope_sweep.__enter__()

    def make_rec_body(gather_cols):
        def rec_body(r, c):
            @pl.when(((r & 127) == 0) & ((r >> 7) >= 2))
            def _():
                pltpu.make_async_copy(
                    stbuf.at[pl.ds(0, 128)], stage_hbm.at[stag2.at[0]], ssem).wait()

            idv = sid[pl.ds(r, _L)][0]
            rrv = jnp.zeros((_L,), jnp.int32) + (idv & 127)
            sbv = jnp.zeros((_L,), jnp.int32) + (r & 255)
            for k in range(EMBED_DIM // _L):
                v = gather_cols(k * _L + lanes, rrv)
                plsc.store_scatter(stbuf, [sbv, k * _L + lanes], v)

            @pl.when((r & 127) == 127)
            def _():
                ch = r >> 7
                pltpu.make_async_copy(
                    stbuf.at[pl.ds((ch & 1) * 128, 128)],
                    stage_hbm.at[stag2.at[ch]], ssem).start()

            return c

        return rec_body

    def blk_body(j, c):
        gb = lo + j
        is_edge = gb == _EDGE
        slot = j & (_NRING - 1)
        bsv = bstart[pl.ds(j, _L)]

        @pl.when(is_edge)
        def _():
            lax.fori_loop(bsv[0], bsv[1], make_rec_body(
                lambda dv, rrv: plsc.load_gather(eblk, [dv, rrv])), 0)

        @pl.when(jnp.logical_not(is_edge))
        def _():
            pltpu.make_async_copy(
                ent_t_hbm.at[:, pl.ds(0, 128)], ring.at[0], dsem).wait()
            slotv = jnp.zeros((_L,), jnp.int32) + slot
            lax.fori_loop(bsv[0], bsv[1], make_rec_body(
                lambda dv, rrv: plsc.load_gather(ring, [slotv, dv, rrv])), 0)

        gb2 = lo + j + _NRING

        @pl.when((j + _NRING < cnt) & (gb2 != _EDGE))
        def _():
            pltpu.make_async_copy(
                ent_t_hbm.at[:, pl.ds(gb2 * 128, 128)],
                ring.at[slot], dsem).start()

        return c

    lax.fori_loop(0, cnt, blk_body, 0)

    @pl.when((nrec & 127) != 0)
    def _():
        ch = nrec >> 7
        pltpu.make_async_copy(
            stbuf.at[pl.ds((ch & 1) * 128, 128)],
            stage_hbm.at[stag2.at[ch]], ssem).start()

    total_fired = (nrec + 127) >> 7
    drained = jnp.maximum(((nrec - 1) >> 7) - 1, 0)

    def d_body(i, c):
        pltpu.make_async_copy(
            stbuf.at[pl.ds(0, 128)], stage_hbm.at[stag2.at[0]], ssem).wait()
        return c

    lax.fori_loop(0, total_fired - drained, d_body, 0)
    _scope_sweep.__exit__(None, None, None)


@functools.partial(
    pl.kernel,
    mesh=_mesh,
    out_type=jax.ShapeDtypeStruct((BATCH,), jnp.float32),
    scratch_types=[
        pltpu.VMEM((8, EMBED_DIM, 128), jnp.float32),
        pltpu.VMEM((512,), jnp.int32),
        pltpu.VMEM((64, 128), jnp.float32),
        pltpu.VMEM((64, 128), jnp.float32),
        pltpu.VMEM((64, 128), jnp.float32),
        pltpu.VMEM((64, 128), jnp.float32),
        pltpu.VMEM((_BPW,), jnp.float32),
        pltpu.SemaphoreType.DMA,
    ],
    compiler_params=pltpu.CompilerParams(needs_layout_passes=False),
)
def _score(stage_hbm, rel_t_hbm, relidx_hbm, out_hbm,
           relbuf, ridx, hbufa, hbufb, tbufa, tbufb, outv, sem):
    wid = lax.axis_index("s") * _NC + lax.axis_index("c")
    base = wid * _BPW
    lanes = lax.iota(jnp.int32, _L)

    copies = [pltpu.async_copy(rel_t_hbm.at[:, pl.ds(k * 128, 128)],
                               relbuf.at[k], sem) for k in range(8)]
    pltpu.sync_copy(relidx_hbm.at[pl.ds(base, _BPW)], ridx)

    npass = _BPW // 64
    hbufs, tbufs = [hbufa, hbufb], [tbufa, tbufb]

    def fire(p):
        return (pltpu.async_copy(stage_hbm.at[pl.ds(base + p * 64, 64)],
                                 hbufs[p % 2], sem),
                pltpu.async_copy(stage_hbm.at[pl.ds(BATCH + base + p * 64, 64)],
                                 tbufs[p % 2], sem))

    pend = {0: fire(0)}
    for c in copies:
        c.wait()

    for p in range(npass):
        hbuf, tbuf = hbufs[p % 2], tbufs[p % 2]
        ch, ct = pend.pop(p)
        ch.wait()
        ct.wait()
        if p + 1 < npass:
            pend[p + 1] = fire(p + 1)

        def g_body(g, c, p=p, hbuf=hbuf, tbuf=tbuf):
            relv = ridx[pl.ds(p * 64 + g * _L, _L)]
            jv = relv >> 7
            rv = relv & 127
            row = g * _L + lanes

            def d_body(d, acc):
                dd = jnp.zeros((_L,), jnp.int32) + d
                h = plsc.load_gather(hbuf, [row, dd])
                t = plsc.load_gather(tbuf, [row, dd])
                r = plsc.load_gather(relbuf, [jv, dd, rv])
                return acc + jnp.abs(h + r - t)

            acc = lax.fori_loop(0, EMBED_DIM, d_body,
                                jnp.zeros((_L,), jnp.float32), unroll=8)
            outv[pl.ds(p * 64 + g * _L, _L)] = acc
            return c

        lax.fori_loop(0, 64 // _L, g_body, 0)

    pltpu.sync_copy(outv, out_hbm.at[pl.ds(base, _BPW)])


def kernel(entity_emb, relation_emb, head, relation, tail):
    ent_t = entity_emb.T
    ent_edge = entity_emb[_EDGE * 128:].T
    rel_t = jnp.pad(relation_emb, ((0, 24), (0, 0))).T
    staging = _sweep(ent_t, ent_edge,
                     head.astype(jnp.int32), tail.astype(jnp.int32))
    return _score(staging, rel_t, relation.astype(jnp.int32))
